# Initial kernel scaffold; baseline (speedup 1.0000x reference)
import jax
import jax.numpy as jnp
from jax.experimental import pallas as pl


def kernel(x, routing_context, gate_w, ctx_w):
    raise NotImplementedError("write your pallas kernel here")


# speedup vs baseline: 3.0778x; 3.0778x over previous
import functools

import jax
import jax.numpy as jnp
from jax.experimental import pallas as pl
from jax.experimental.pallas import tpu as pltpu


def _router_body(x_ref, rc_ref, gw_ref, cw_ref,
                 idx_ref, w_ref, aux_ref,
                 ctx_scr, imp_scr, load_scr,
                 *, n_rows, n_experts, n_b, n_j):
    b = pl.program_id(0)
    j = pl.program_id(1)

    @pl.when(jnp.logical_and(b == 0, j == 0))
    def _init():
        ctx_scr[...] = jax.lax.dot_general(
            rc_ref[...], cw_ref[...], (((1,), (1,)), ((), ())),
            preferred_element_type=jnp.float32)
        imp_scr[...] = jnp.zeros_like(imp_scr)
        load_scr[...] = jnp.zeros_like(load_scr)

    xb = x_ref[0]
    routing = xb + ctx_scr[pl.ds(b, 1), :]
    logits = jax.lax.dot_general(
        routing, gw_ref[...], (((1,), (1,)), ((), ())),
        preferred_element_type=jnp.float32)

    m1 = jnp.max(logits, axis=1, keepdims=True)
    i1 = jnp.argmax(logits, axis=1).astype(jnp.int32)
    iota = jax.lax.broadcasted_iota(jnp.int32, (n_rows, n_experts), 1)
    onehot1 = iota == i1[:, None]
    masked = jnp.where(onehot1, jnp.float32(-jnp.inf), logits)
    m2 = jnp.max(masked, axis=1, keepdims=True)
    i2 = jnp.argmax(masked, axis=1).astype(jnp.int32)
    onehot2 = iota == i2[:, None]

    t = jnp.exp(m2 - m1)
    s = 1.0 + t
    idx_ref[...] = jnp.concatenate([i1[:, None], i2[:, None]], axis=1)
    w_ref[...] = jnp.concatenate([1.0 / s, t / s], axis=1)

    p = jnp.exp(logits - m1)
    probs = p / jnp.sum(p, axis=1, keepdims=True)
    imp_scr[...] += jnp.sum(probs, axis=0, keepdims=True)
    load_scr[...] += jnp.sum(onehot1.astype(jnp.float32) +
                             onehot2.astype(jnp.float32), axis=0, keepdims=True)

    @pl.when(jnp.logical_and(b == n_b - 1, j == n_j - 1))
    def _finish():
        total = jnp.float32(n_b * n_j * n_rows)
        aux_ref[...] = (jnp.float32(n_experts) *
                        jnp.sum(imp_scr[...] * load_scr[...], axis=1,
                                keepdims=True) / (total * total))


def kernel(x, routing_context, gate_w, ctx_w):
    b, n, c = x.shape
    e = gate_w.shape[0]
    t = b * n
    R = 512
    n_j = n // R

    body = functools.partial(_router_body, n_rows=R, n_experts=e, n_b=b, n_j=n_j)
    top_idx, top_w, aux = pl.pallas_call(
        body,
        grid=(b, n_j),
        in_specs=[
            pl.BlockSpec((1, R, c), lambda bi, ji: (bi, ji, 0)),
            pl.BlockSpec((b, c), lambda bi, ji: (0, 0)),
            pl.BlockSpec((e, c), lambda bi, ji: (0, 0)),
            pl.BlockSpec((c, c), lambda bi, ji: (0, 0)),
        ],
        out_specs=[
            pl.BlockSpec((R, 2), lambda bi, ji: (bi * (n // R) + ji, 0)),
            pl.BlockSpec((R, 2), lambda bi, ji: (bi * (n // R) + ji, 0)),
            pl.BlockSpec((1, 1), lambda bi, ji: (0, 0)),
        ],
        out_shape=[
            jax.ShapeDtypeStruct((t, 2), jnp.int32),
            jax.ShapeDtypeStruct((t, 2), jnp.float32),
            jax.ShapeDtypeStruct((1, 1), jnp.float32),
        ],
        scratch_shapes=[
            pltpu.VMEM((b, c), jnp.float32),
            pltpu.VMEM((1, e), jnp.float32),
            pltpu.VMEM((1, e), jnp.float32),
        ],
    )(x, routing_context, gate_w, ctx_w)
    return (top_idx, top_w, jnp.reshape(aux, ()))
